# baseline (speedup 1.0000x reference)
import functools

import jax
import jax.numpy as jnp
from jax import lax
from jax.experimental import pallas as pl
from jax.experimental.pallas import tpu as pltpu
from jax.experimental.pallas import tpu_sc as plsc

_L = 16
_NC = 2
_NS = 16
_NW = _NC * _NS

_X0, _X1 = -51.2, 51.2
_Y0, _Y1 = -51.2, 51.2
_Z0, _Z1 = -5.0, 3.0
_DX = (_X1 - _X0) / 200.0
_DY = (_Y1 - _Y0) / 200.0
_RND = 8388608.0
_NROWS = 20
_UNROLL = 8
import numpy as _np
_IDX = float(_np.float32(1.0) / _np.float32(_DX))
_IDY = float(_np.float32(1.0) / _np.float32(_DY))


@functools.cache
def _make_sc_call(npad: int):
    b_per_w = npad // _NW
    nvec = b_per_w // _L
    mesh = plsc.VectorSubcoreMesh(core_axis_name="c", subcore_axis_name="s",
                                  num_cores=_NC, num_subcores=_NS)

    @functools.partial(
        pl.kernel,
        out_type=[
            jax.ShapeDtypeStruct((3 * npad,), jnp.int32),
        ],
        mesh=mesh,
        scratch_types=[
            pltpu.VMEM((b_per_w,), jnp.float32),
            pltpu.VMEM((b_per_w,), jnp.float32),
            pltpu.VMEM((b_per_w,), jnp.float32),
            pltpu.VMEM((_NROWS, _L), jnp.float32),
            pltpu.VMEM((b_per_w,), jnp.int32),
            pltpu.VMEM((b_per_w,), jnp.int32),
            pltpu.VMEM((b_per_w,), jnp.int32),
        ],
    )
    def call(uvd_hbm, p_hbm, out_hbm,
             u_v, v_v, d_v, p_v, m_v, i_v, j_v):
        wid = lax.axis_index("s") * _NC + lax.axis_index("c")
        base = wid * b_per_w
        pltpu.sync_copy(uvd_hbm.at[pl.ds(base, b_per_w)], u_v)
        pltpu.sync_copy(uvd_hbm.at[pl.ds(npad + base, b_per_w)], v_v)
        pltpu.sync_copy(uvd_hbm.at[pl.ds(2 * npad + base, b_per_w)], d_v)
        pltpu.sync_copy(p_hbm, p_v)

        def body(k, carry):
            cx, cy, ifx, ify = p_v[0], p_v[1], p_v[2], p_v[3]
            r00, r01, r02 = p_v[4], p_v[5], p_v[6]
            r10, r11, r12 = p_v[7], p_v[8], p_v[9]
            r20, r21, r22 = p_v[10], p_v[11], p_v[12]
            t0, t1, t2 = p_v[13], p_v[14], p_v[15]
            umax, vmax = p_v[16], p_v[17]
            jmax, imax = p_v[18], p_v[19]
            for t_ in range(_UNROLL):
                body_one(k * _UNROLL + t_, cx, cy, ifx, ify,
                         r00, r01, r02, r10, r11, r12, r20, r21, r22,
                         t0, t1, t2, umax, vmax, jmax, imax)
            return carry

        def body_one(kv, cx, cy, ifx, ify,
                     r00, r01, r02, r10, r11, r12, r20, r21, r22,
                     t0, t1, t2, umax, vmax, jmax, imax):
            off = kv * _L
            u_raw = u_v[pl.ds(off, _L)]
            v_raw = v_v[pl.ds(off, _L)]
            dep = d_v[pl.ds(off, _L)]

            one = jnp.full((16,), 1, jnp.int32)
            zero = jnp.full((16,), 0, jnp.int32)
            u05 = u_raw + 0.5
            v05 = v_raw + 0.5
            ui = u05.astype(jnp.int32)
            vi = v05.astype(jnp.int32)
            utie = (ui.astype(jnp.float32) == u05) & ((ui & 1) == 1)
            vtie = (vi.astype(jnp.float32) == v05) & ((vi & 1) == 1)
            ui = ui - jnp.where(utie, one, zero)
            vi = vi - jnp.where(vtie, one, zero)
            u = jnp.minimum(jnp.maximum(ui.astype(jnp.float32), 0.0), umax)
            v = jnp.minimum(jnp.maximum(vi.astype(jnp.float32), 0.0), vmax)

            a = (u - cx) * ifx
            b = (v - cy) * ify
            s = a * a + b * b + 1.0
            hs = 0.5 * s
            r = 1.0 / s
            r = r * (1.5 - hs * r * r)
            r = r * (1.5 - hs * r * r)
            r = r * (1.5 - hs * r * r)
            r = r * (1.5 - hs * r * r)
            def _bf16(x_):
                c_ = x_ * 65537.0
                return c_ - (c_ - x_)
            dx = _bf16(a * r)
            dy = _bf16(b * r)
            dz = _bf16(r)

            x = t0 + (r00 * dx + r01 * dy + r02 * dz) * dep
            y = t1 + (r10 * dx + r11 * dy + r12 * dz) * dep
            z = t2 + (r20 * dx + r21 * dy + r22 * dz) * dep

            m = ((x >= _X0) & (x < _X1) & (y >= _Y0) & (y < _Y1)
                 & (z >= _Z0) & (z < _Z1))

            jf = jnp.minimum(jnp.maximum((x - _X0) * _IDX, 0.0), jmax)
            if_ = jnp.minimum(jnp.maximum((y - _Y0) * _IDY, 0.0), imax)
            jc = jf.astype(jnp.int32)
            ic = if_.astype(jnp.int32)
            neg1 = jnp.full((16,), -1, jnp.int32)

            m_v[pl.ds(off, _L)] = jnp.where(m, one, zero)
            i_v[pl.ds(off, _L)] = jnp.where(m, ic, neg1)
            j_v[pl.ds(off, _L)] = jnp.where(m, jc, neg1)

        lax.fori_loop(0, nvec // _UNROLL, body, 0)

        pltpu.sync_copy(m_v, out_hbm.at[pl.ds(base, b_per_w)])
        pltpu.sync_copy(i_v, out_hbm.at[pl.ds(npad + base, b_per_w)])
        pltpu.sync_copy(j_v, out_hbm.at[pl.ds(2 * npad + base, b_per_w)])

    return call


def kernel(pix_uv, depth_mu, K, T_cam2ego, H, W, Hb, Wb):
    N = pix_uv.shape[0]
    chunk = _NW * _L * _UNROLL
    npad = ((N + chunk - 1) // chunk) * chunk

    uvd = jnp.concatenate([
        jnp.pad(pix_uv[:, 0], (0, npad - N)),
        jnp.pad(pix_uv[:, 1], (0, npad - N)),
        jnp.pad(depth_mu, (0, npad - N)),
    ])

    R = T_cam2ego[:3, :3]
    t = T_cam2ego[:3, 3]
    vals = jnp.stack([
        K[0, 2], K[1, 2], 1.0 / K[0, 0], 1.0 / K[1, 1],
        R[0, 0], R[0, 1], R[0, 2],
        R[1, 0], R[1, 1], R[1, 2],
        R[2, 0], R[2, 1], R[2, 2],
        t[0], t[1], t[2],
        jnp.float32(W - 1), jnp.float32(H - 1),
        jnp.float32(Wb - 1), jnp.float32(Hb - 1),
    ]).astype(jnp.float32)
    params = jnp.broadcast_to(vals[:, None], (_NROWS, _L))

    out = _make_sc_call(npad)(uvd, params)[0]

    m = out[:N].astype(bool)
    ij = jnp.stack([out[npad:npad + N], out[2 * npad:2 * npad + N]], axis=-1)
    return (m, ij)


# speedup vs baseline: 1.0117x; 1.0117x over previous
import functools

import jax
import jax.numpy as jnp
from jax import lax
from jax.experimental import pallas as pl
from jax.experimental.pallas import tpu as pltpu
from jax.experimental.pallas import tpu_sc as plsc

_L = 16
_NC = 2
_NS = 16
_NW = _NC * _NS

_X0, _X1 = -51.2, 51.2
_Y0, _Y1 = -51.2, 51.2
_Z0, _Z1 = -5.0, 3.0
_DX = (_X1 - _X0) / 200.0
_DY = (_Y1 - _Y0) / 200.0
_RND = 8388608.0
_NROWS = 20
_UNROLL = 8
import numpy as _np
_IDX = float(_np.float32(1.0) / _np.float32(_DX))
_IDY = float(_np.float32(1.0) / _np.float32(_DY))


@functools.cache
def _make_sc_call(npad: int):
    b_per_w = npad // _NW
    nvec = b_per_w // _L
    mesh = plsc.VectorSubcoreMesh(core_axis_name="c", subcore_axis_name="s",
                                  num_cores=_NC, num_subcores=_NS)

    @functools.partial(
        pl.kernel,
        out_type=[
            jax.ShapeDtypeStruct((npad,), jnp.int32),
            jax.ShapeDtypeStruct((npad,), jnp.int32),
            jax.ShapeDtypeStruct((npad,), jnp.int32),
        ],
        mesh=mesh,
        scratch_types=[
            pltpu.VMEM((b_per_w,), jnp.float32),
            pltpu.VMEM((b_per_w,), jnp.float32),
            pltpu.VMEM((b_per_w,), jnp.float32),
            pltpu.VMEM((_NROWS, _L), jnp.float32),
            pltpu.VMEM((b_per_w,), jnp.int32),
            pltpu.VMEM((b_per_w,), jnp.int32),
            pltpu.VMEM((b_per_w,), jnp.int32),
        ],
    )
    def call(u_hbm, v_hbm, d_hbm, p_hbm, m_hbm, i_hbm, j_hbm,
             u_v, v_v, d_v, p_v, m_v, i_v, j_v):
        wid = lax.axis_index("s") * _NC + lax.axis_index("c")
        base = wid * b_per_w
        pltpu.sync_copy(u_hbm.at[pl.ds(base, b_per_w)], u_v)
        pltpu.sync_copy(v_hbm.at[pl.ds(base, b_per_w)], v_v)
        pltpu.sync_copy(d_hbm.at[pl.ds(base, b_per_w)], d_v)
        pltpu.sync_copy(p_hbm, p_v)

        def body(k, carry):
            cx, cy, ifx, ify = p_v[0], p_v[1], p_v[2], p_v[3]
            r00, r01, r02 = p_v[4], p_v[5], p_v[6]
            r10, r11, r12 = p_v[7], p_v[8], p_v[9]
            r20, r21, r22 = p_v[10], p_v[11], p_v[12]
            t0, t1, t2 = p_v[13], p_v[14], p_v[15]
            umax, vmax = p_v[16], p_v[17]
            jmax, imax = p_v[18], p_v[19]
            for t_ in range(_UNROLL):
                body_one(k * _UNROLL + t_, cx, cy, ifx, ify,
                         r00, r01, r02, r10, r11, r12, r20, r21, r22,
                         t0, t1, t2, umax, vmax, jmax, imax)
            return carry

        def body_one(kv, cx, cy, ifx, ify,
                     r00, r01, r02, r10, r11, r12, r20, r21, r22,
                     t0, t1, t2, umax, vmax, jmax, imax):
            off = kv * _L
            u_raw = u_v[pl.ds(off, _L)]
            v_raw = v_v[pl.ds(off, _L)]
            dep = d_v[pl.ds(off, _L)]

            one = jnp.full((16,), 1, jnp.int32)
            zero = jnp.full((16,), 0, jnp.int32)
            u05 = u_raw + 0.5
            v05 = v_raw + 0.5
            ui = u05.astype(jnp.int32)
            vi = v05.astype(jnp.int32)
            utie = (ui.astype(jnp.float32) == u05) & ((ui & 1) == 1)
            vtie = (vi.astype(jnp.float32) == v05) & ((vi & 1) == 1)
            ui = ui - jnp.where(utie, one, zero)
            vi = vi - jnp.where(vtie, one, zero)
            u = jnp.minimum(jnp.maximum(ui.astype(jnp.float32), 0.0), umax)
            v = jnp.minimum(jnp.maximum(vi.astype(jnp.float32), 0.0), vmax)

            a = (u - cx) * ifx
            b = (v - cy) * ify
            s = a * a + b * b + 1.0
            hs = 0.5 * s
            r = 1.0 / s
            r = r * (1.5 - hs * r * r)
            r = r * (1.5 - hs * r * r)
            r = r * (1.5 - hs * r * r)
            r = r * (1.5 - hs * r * r)
            def _bf16(x_):
                c_ = x_ * 65537.0
                return c_ - (c_ - x_)
            dx = _bf16(a * r)
            dy = _bf16(b * r)
            dz = _bf16(r)

            x = t0 + (r00 * dx + r01 * dy + r02 * dz) * dep
            y = t1 + (r10 * dx + r11 * dy + r12 * dz) * dep
            z = t2 + (r20 * dx + r21 * dy + r22 * dz) * dep

            m = ((x >= _X0) & (x < _X1) & (y >= _Y0) & (y < _Y1)
                 & (z >= _Z0) & (z < _Z1))

            jf = jnp.minimum(jnp.maximum((x - _X0) * _IDX, 0.0), jmax)
            if_ = jnp.minimum(jnp.maximum((y - _Y0) * _IDY, 0.0), imax)
            jc = jf.astype(jnp.int32)
            ic = if_.astype(jnp.int32)
            neg1 = jnp.full((16,), -1, jnp.int32)

            m_v[pl.ds(off, _L)] = jnp.where(m, one, zero)
            i_v[pl.ds(off, _L)] = jnp.where(m, ic, neg1)
            j_v[pl.ds(off, _L)] = jnp.where(m, jc, neg1)

        lax.fori_loop(0, nvec // _UNROLL, body, 0)

        pltpu.sync_copy(m_v, m_hbm.at[pl.ds(base, b_per_w)])
        pltpu.sync_copy(i_v, i_hbm.at[pl.ds(base, b_per_w)])
        pltpu.sync_copy(j_v, j_hbm.at[pl.ds(base, b_per_w)])

    return call


def kernel(pix_uv, depth_mu, K, T_cam2ego, H, W, Hb, Wb):
    N = pix_uv.shape[0]
    chunk = _NW * _L * _UNROLL
    npad = ((N + chunk - 1) // chunk) * chunk

    u_pad = jnp.pad(pix_uv[:, 0], (0, npad - N))
    v_pad = jnp.pad(pix_uv[:, 1], (0, npad - N))
    d_pad = jnp.pad(depth_mu, (0, npad - N))

    R = T_cam2ego[:3, :3]
    t = T_cam2ego[:3, 3]
    vals = jnp.stack([
        K[0, 2], K[1, 2], 1.0 / K[0, 0], 1.0 / K[1, 1],
        R[0, 0], R[0, 1], R[0, 2],
        R[1, 0], R[1, 1], R[1, 2],
        R[2, 0], R[2, 1], R[2, 2],
        t[0], t[1], t[2],
        jnp.float32(W - 1), jnp.float32(H - 1),
        jnp.float32(Wb - 1), jnp.float32(Hb - 1),
    ]).astype(jnp.float32)
    params = jnp.broadcast_to(vals[:, None], (_NROWS, _L))

    m_i32, i_arr, j_arr = _make_sc_call(npad)(u_pad, v_pad, d_pad, params)

    m = m_i32[:N].astype(bool)
    ij = jnp.stack([i_arr[:N], j_arr[:N]], axis=-1)
    return (m, ij)
